# Initial kernel scaffold; baseline (speedup 1.0000x reference)
import jax
import jax.numpy as jnp
from jax.experimental import pallas as pl


def kernel(init_ent_emb, node_id, edge_index, edge_type, init_rel_emb, W_neigh, W_self):
    raise NotImplementedError("write your pallas kernel here")


# speedup vs baseline: 2.9479x; 2.9479x over previous
import functools

import jax
import jax.numpy as jnp
from jax import lax
from jax.experimental import pallas as pl
from jax.experimental.pallas import tpu as pltpu
from jax.experimental.pallas import tpu_sc as plsc

N = 10000
E = 320000
D = 128
R = 200
C = 0.01
SQRT_C = 0.1
RRELU_SLOPE = (1.0 / 8.0 + 1.0 / 3.0) / 2.0

NC = 2
NS = 16
NW = NC * NS
N_PAD = 10240
R_PAD = 256
R_SC = 50
E_PAD = 327680
EPT = E_PAD // NS
EPW = E_PAD // NW
CHUNK = 128
CH_E = 64
NCH_E = EPW // CH_E
NCH_C = EPT // CHUNK
ROW_STRIPE = N_PAD // NS
CNT_W = N_PAD * R_SC
CNT_STRIPE = CNT_W // NS


def _sc_cnt_body(dst_hbm, et_hbm, base_hbm, zeros_hbm, out_hbm,
                 dst_v, et_v, base_v, idx_v, val_v, acc):
    c = lax.axis_index("c")
    s = lax.axis_index("s")
    pltpu.sync_copy(zeros_hbm.at[pl.ds(s * CNT_STRIPE, CNT_STRIPE)],
                    acc.at[pl.ds(s * CNT_STRIPE, CNT_STRIPE)])
    pltpu.sync_copy(dst_hbm.at[s], dst_v)
    pltpu.sync_copy(et_hbm.at[s], et_v)
    pltpu.sync_copy(base_hbm, base_v)
    plsc.subcore_barrier()
    lo_vec = base_v[...] + c * R_SC

    def chunk_body(j, carry):
        for i in range(CHUNK // 16):
            sl = pl.ds(j * CHUNK + i * 16, 16)
            d = dst_v[sl]
            rel = et_v[sl] - lo_vec
            mr = (rel >= 0) & (rel < R_SC)
            idx_v[0, pl.ds(i * 16, 16)] = jnp.where(mr, d * R_SC + rel, 0)
            val_v[0, pl.ds(i * 16, 16)] = jnp.where(mr, 1.0, 0.0)
        pltpu.sync_copy(val_v.at[0], acc.at[idx_v.at[0]], add=True)
        return carry

    lax.fori_loop(0, NCH_C, chunk_body, 0)
    plsc.subcore_barrier()
    pltpu.sync_copy(acc.at[pl.ds(s * CNT_STRIPE, CNT_STRIPE)],
                    out_hbm.at[c, pl.ds(s * CNT_STRIPE, CNT_STRIPE)])


def _sc_edge_body(hw_hbm, pack_hbm, zeros_hbm, out_hbm,
                  pack_v, gidx0, sidx0, gidx1, sidx1,
                  rows0_v, rows1_v, acc, gsem0, gsem1):
    c = lax.axis_index("c")
    s = lax.axis_index("s")
    wid = s * NC + c
    pltpu.sync_copy(zeros_hbm.at[pl.ds(s * ROW_STRIPE, ROW_STRIPE)],
                    acc.at[pl.ds(s * ROW_STRIPE, ROW_STRIPE)])
    pltpu.sync_copy(pack_hbm.at[wid], pack_v)
    plsc.subcore_barrier()

    def unpack(j, gidx, sidx):
        for k in range(CH_E // 16):
            sl = pl.ds(j * CH_E + k * 16, 16)
            w = pack_v[sl]
            gidx[0, pl.ds(k * 16, 16)] = w & 16383
            sidx[0, pl.ds(k * 16, 16)] = jax.lax.shift_right_logical(w, 14)

    unpack(0, gidx0, sidx0)
    pltpu.async_copy(hw_hbm.at[gidx0.at[0]], rows0_v, gsem0)

    def pair_body(jj, carry):
        j0 = jj * 2
        j1 = j0 + 1
        unpack(j1, gidx1, sidx1)
        pltpu.async_copy(hw_hbm.at[gidx1.at[0]], rows1_v, gsem1)
        pltpu.make_async_copy(hw_hbm.at[gidx0.at[0]], rows0_v, gsem0).wait()
        pltpu.sync_copy(rows0_v, acc.at[sidx0.at[0]], add=True)
        unpack(jnp.minimum(j0 + 2, NCH_E - 1), gidx0, sidx0)
        pltpu.async_copy(hw_hbm.at[gidx0.at[0]], rows0_v, gsem0)
        pltpu.make_async_copy(hw_hbm.at[gidx1.at[0]], rows1_v, gsem1).wait()
        pltpu.sync_copy(rows1_v, acc.at[sidx1.at[0]], add=True)
        return carry

    lax.fori_loop(0, NCH_E // 2, pair_body, 0)
    pltpu.make_async_copy(hw_hbm.at[gidx0.at[0]], rows0_v, gsem0).wait()
    plsc.subcore_barrier()
    pltpu.sync_copy(acc.at[pl.ds(s * ROW_STRIPE, ROW_STRIPE)],
                    out_hbm.at[c, pl.ds(s * ROW_STRIPE, ROW_STRIPE)])


@functools.cache
def _sc_kernels():
    mesh = plsc.VectorSubcoreMesh(core_axis_name="c", subcore_axis_name="s")
    sc_cnt = functools.partial(
        pl.kernel,
        out_type=jax.ShapeDtypeStruct((NC, CNT_W), jnp.float32),
        mesh=mesh,
        scratch_types=[
            pltpu.VMEM((EPT,), jnp.int32),
            pltpu.VMEM((EPT,), jnp.int32),
            pltpu.VMEM((16,), jnp.int32),
            pltpu.VMEM((1, CHUNK), jnp.int32),
            pltpu.VMEM((1, CHUNK), jnp.float32),
            pltpu.VMEM_SHARED((CNT_W,), jnp.float32),
        ],
    )(_sc_cnt_body)
    sc_edge = functools.partial(
        pl.kernel,
        out_type=jax.ShapeDtypeStruct((NC, N_PAD, D), jnp.float32),
        mesh=mesh,
        scratch_types=[
            pltpu.VMEM((EPW,), jnp.int32),
            pltpu.VMEM((1, CH_E), jnp.int32),
            pltpu.VMEM((1, CH_E), jnp.int32),
            pltpu.VMEM((1, CH_E), jnp.int32),
            pltpu.VMEM((1, CH_E), jnp.int32),
            pltpu.VMEM((CH_E, D), jnp.float32),
            pltpu.VMEM((CH_E, D), jnp.float32),
            pltpu.VMEM_SHARED((N_PAD, D), jnp.float32),
            pltpu.SemaphoreType.DMA,
            pltpu.SemaphoreType.DMA,
        ],
    )(_sc_edge_body)
    return sc_cnt, sc_edge


BLK = 1024
GRID = N_PAD // BLK


def _leaky(x):
    return jnp.where(x >= 0, x, RRELU_SLOPE * x)


def _tc_pre_body(x_ref, wn_ref, ws_ref, ht_ref, hw_ref, lp_ref):
    x = x_ref[...]
    n = jnp.sqrt(jnp.sum(x * x, axis=-1, keepdims=True))
    n = jnp.maximum(n, 1e-10)
    y = jnp.clip(SQRT_C * n, -1.0 + 1e-7, 1.0 - 1e-7)
    at = 0.5 * jnp.log((1.0 + y) / (1.0 - y))
    ht = at * x / (SQRT_C * n)
    ht_ref[...] = ht
    hw_ref[...] = jnp.dot(ht, wn_ref[...], preferred_element_type=jnp.float32)
    lp_ref[...] = jnp.dot(ht, ws_ref[...], preferred_element_type=jnp.float32)


def _tc_combine_body(p0_ref, p1_ref, cnt_ref, rel_ref, wn_ref, lp_ref,
                     wn2_ref, ws2_ref, hw2_ref, lp2_ref):
    relw = jnp.dot(rel_ref[...], wn_ref[...], preferred_element_type=jnp.float32)
    cnt = cnt_ref[...]
    aggrel = jnp.dot(cnt, relw, preferred_element_type=jnp.float32)
    deg = jnp.maximum(jnp.sum(cnt, axis=-1, keepdims=True), 1.0)
    pre = (p0_ref[...] + p1_ref[...] + aggrel) / deg + lp_ref[...]
    ht = _leaky(pre)
    hw2_ref[...] = jnp.dot(ht, wn2_ref[...], preferred_element_type=jnp.float32)
    lp2_ref[...] = jnp.dot(ht, ws2_ref[...], preferred_element_type=jnp.float32)


def _tc_final_body(p0_ref, p1_ref, cnt_ref, rel_ref, wn_ref, lp_ref, out_ref):
    relw = jnp.dot(rel_ref[...], wn_ref[...], preferred_element_type=jnp.float32)
    cnt = cnt_ref[...]
    aggrel = jnp.dot(cnt, relw, preferred_element_type=jnp.float32)
    deg = jnp.maximum(jnp.sum(cnt, axis=-1, keepdims=True), 1.0)
    pre = (p0_ref[...] + p1_ref[...] + aggrel) / deg + lp_ref[...]
    ht = _leaky(pre)
    n = jnp.sqrt(jnp.sum(ht * ht, axis=-1, keepdims=True))
    n = jnp.maximum(n, 1e-10)
    out_ref[...] = jnp.tanh(SQRT_C * n) * ht / (SQRT_C * n)


_row_spec = pl.BlockSpec((BLK, D), lambda i: (i, 0))
_cnt_spec = pl.BlockSpec((BLK, R_PAD), lambda i: (i, 0))
_full_spec = pl.BlockSpec((D, D), lambda i: (0, 0))
_rel_spec = pl.BlockSpec((R_PAD, D), lambda i: (0, 0))
_rowD = jax.ShapeDtypeStruct((N_PAD, D), jnp.float32)


def _tc_pre(x, wn, ws):
    return pl.pallas_call(
        _tc_pre_body,
        grid=(GRID,),
        in_specs=[_row_spec, _full_spec, _full_spec],
        out_specs=[_row_spec, _row_spec, _row_spec],
        out_shape=[_rowD, _rowD, _rowD],
    )(x, wn, ws)


def _tc_combine(p0, p1, cnt, rel, wn, lp, wn2, ws2):
    return pl.pallas_call(
        _tc_combine_body,
        grid=(GRID,),
        in_specs=[_row_spec, _row_spec, _cnt_spec, _rel_spec, _full_spec,
                  _row_spec, _full_spec, _full_spec],
        out_specs=[_row_spec, _row_spec],
        out_shape=[_rowD, _rowD],
    )(p0, p1, cnt, rel, wn, lp, wn2, ws2)


def _tc_final(p0, p1, cnt, rel, wn, lp):
    return pl.pallas_call(
        _tc_final_body,
        grid=(GRID,),
        in_specs=[_row_spec, _row_spec, _cnt_spec, _rel_spec, _full_spec,
                  _row_spec],
        out_specs=_row_spec,
        out_shape=_rowD,
    )(p0, p1, cnt, rel, wn, lp)


def kernel(init_ent_emb, node_id, edge_index, edge_type, init_rel_emb,
           W_neigh, W_self):
    h = jnp.take(init_ent_emb, node_id, axis=0)
    h_pad = jnp.pad(h, ((0, N_PAD - N), (0, 0)))
    rel_pad = jnp.pad(init_rel_emb, ((0, R_PAD - R), (0, 0)))

    src = jnp.pad(edge_index[0], (0, E_PAD - E))
    dst = jnp.pad(edge_index[1], (0, E_PAD - E), constant_values=N)
    et = jnp.pad(edge_type, (0, E_PAD - E))

    dst2 = dst.reshape(NS, EPT)
    et2 = et.reshape(NS, EPT)
    pack2 = (src | (dst << 14)).reshape(NW, EPW)

    zeros_flat = jnp.zeros((CNT_W,), jnp.float32)
    zeros_rows = jnp.zeros((N_PAD, D), jnp.float32)
    base0 = jnp.zeros((16,), jnp.int32)
    base1 = jnp.full((16,), 2 * R_SC, jnp.int32)

    _sc_cnt, _sc_edge = _sc_kernels()
    cnt_a = _sc_cnt(dst2, et2, base0, zeros_flat)
    cnt_b = _sc_cnt(dst2, et2, base1, zeros_flat)
    cnt = jnp.concatenate(
        [cnt_a[0].reshape(N_PAD, R_SC), cnt_a[1].reshape(N_PAD, R_SC),
         cnt_b[0].reshape(N_PAD, R_SC), cnt_b[1].reshape(N_PAD, R_SC)], axis=1)
    cnt = jnp.pad(cnt, ((0, 0), (0, R_PAD - 4 * R_SC)))

    h_tan, hw0, lp0 = _tc_pre(h_pad, W_neigh[0], W_self[0])
    part0 = _sc_edge(hw0, pack2, zeros_rows)
    hw1, lp1 = _tc_combine(part0[0], part0[1], cnt, rel_pad, W_neigh[0], lp0,
                           W_neigh[1], W_self[1])
    part1 = _sc_edge(hw1, pack2, zeros_rows)
    out = _tc_final(part1[0], part1[1], cnt, rel_pad, W_neigh[1], lp1)
    return out[:N]


# speedup vs baseline: 3.0387x; 1.0308x over previous
import functools

import jax
import jax.numpy as jnp
from jax import lax
from jax.experimental import pallas as pl
from jax.experimental.pallas import tpu as pltpu
from jax.experimental.pallas import tpu_sc as plsc

N = 10000
E = 320000
D = 128
R = 200
C = 0.01
SQRT_C = 0.1
RRELU_SLOPE = (1.0 / 8.0 + 1.0 / 3.0) / 2.0

NC = 2
NS = 16
NW = NC * NS
N_PAD = 10240
R_PAD = 256
R_SC = 50
E_PAD = 327680
EPT = E_PAD // NS
EPW = E_PAD // NW
CHUNK = 128
CH_E = 128
NCH_E = EPW // CH_E
NCH_C = EPT // CHUNK
ROW_STRIPE = N_PAD // NS
CNT_W = N_PAD * R_SC
CNT_STRIPE = CNT_W // NS


def _sc_cnt_body(dst_hbm, et_hbm, zeros_hbm, out_hbm,
                 dst_v, et_v, idx_v, val_v, acc, ssem):
    c = lax.axis_index("c")
    s = lax.axis_index("s")
    pltpu.sync_copy(dst_hbm.at[s], dst_v)
    pltpu.sync_copy(et_hbm.at[s], et_v)
    for phase in range(2):
        pltpu.sync_copy(zeros_hbm.at[pl.ds(s * CNT_STRIPE, CNT_STRIPE)],
                        acc.at[pl.ds(s * CNT_STRIPE, CNT_STRIPE)])
        plsc.subcore_barrier()
        lo = phase * 2 * R_SC + c * R_SC

        def chunk_body(jj, carry):
            for b in range(8):
                j = jj * 8 + b
                for i in range(CHUNK // 16):
                    sl = pl.ds(j * CHUNK + i * 16, 16)
                    d = dst_v[sl]
                    rel = et_v[sl] - lo
                    mr = (rel >= 0) & (rel < R_SC)
                    idx_v[b, pl.ds(i * 16, 16)] = jnp.where(mr, d * R_SC + rel, 0)
                    val_v[b, pl.ds(i * 16, 16)] = jnp.where(mr, 1.0, 0.0)
                pltpu.async_copy(val_v.at[b], acc.at[idx_v.at[b]], ssem, add=True)
            for b in range(8):
                pltpu.make_async_copy(val_v.at[b], acc.at[idx_v.at[b]], ssem).wait()
            return carry

        lax.fori_loop(0, NCH_C // 8, chunk_body, 0)
        plsc.subcore_barrier()
        pltpu.sync_copy(acc.at[pl.ds(s * CNT_STRIPE, CNT_STRIPE)],
                        out_hbm.at[phase, c, pl.ds(s * CNT_STRIPE, CNT_STRIPE)])


def _sc_edge_body(hw_hbm, pack_hbm, zeros_hbm, out_hbm,
                  pack_v, gidx0, sidx0, gidx1, sidx1,
                  rows0_v, rows1_v, acc, gsem0, gsem1):
    c = lax.axis_index("c")
    s = lax.axis_index("s")
    wid = s * NC + c
    pltpu.sync_copy(zeros_hbm.at[pl.ds(s * ROW_STRIPE, ROW_STRIPE)],
                    acc.at[pl.ds(s * ROW_STRIPE, ROW_STRIPE)])
    pltpu.sync_copy(pack_hbm.at[wid], pack_v)
    plsc.subcore_barrier()

    def unpack(j, gidx, sidx):
        for k in range(CH_E // 16):
            sl = pl.ds(j * CH_E + k * 16, 16)
            w = pack_v[sl]
            gidx[0, pl.ds(k * 16, 16)] = w & 16383
            sidx[0, pl.ds(k * 16, 16)] = jax.lax.shift_right_logical(w, 14)

    unpack(0, gidx0, sidx0)
    pltpu.async_copy(hw_hbm.at[gidx0.at[0]], rows0_v, gsem0)

    def pair_body(jj, carry):
        j0 = jj * 2
        j1 = j0 + 1
        unpack(j1, gidx1, sidx1)
        pltpu.async_copy(hw_hbm.at[gidx1.at[0]], rows1_v, gsem1)
        pltpu.make_async_copy(hw_hbm.at[gidx0.at[0]], rows0_v, gsem0).wait()
        pltpu.sync_copy(rows0_v, acc.at[sidx0.at[0]], add=True)
        unpack(jnp.minimum(j0 + 2, NCH_E - 1), gidx0, sidx0)
        pltpu.async_copy(hw_hbm.at[gidx0.at[0]], rows0_v, gsem0)
        pltpu.make_async_copy(hw_hbm.at[gidx1.at[0]], rows1_v, gsem1).wait()
        pltpu.sync_copy(rows1_v, acc.at[sidx1.at[0]], add=True)
        return carry

    lax.fori_loop(0, NCH_E // 2, pair_body, 0)
    pltpu.make_async_copy(hw_hbm.at[gidx0.at[0]], rows0_v, gsem0).wait()
    plsc.subcore_barrier()
    pltpu.sync_copy(acc.at[pl.ds(s * ROW_STRIPE, ROW_STRIPE)],
                    out_hbm.at[c, pl.ds(s * ROW_STRIPE, ROW_STRIPE)])


@functools.cache
def _sc_kernels():
    mesh = plsc.VectorSubcoreMesh(core_axis_name="c", subcore_axis_name="s")
    sc_cnt = functools.partial(
        pl.kernel,
        out_type=jax.ShapeDtypeStruct((2, NC, CNT_W), jnp.float32),
        mesh=mesh,
        scratch_types=[
            pltpu.VMEM((EPT,), jnp.int32),
            pltpu.VMEM((EPT,), jnp.int32),
            pltpu.VMEM((8, CHUNK), jnp.int32),
            pltpu.VMEM((8, CHUNK), jnp.float32),
            pltpu.VMEM_SHARED((CNT_W,), jnp.float32),
            pltpu.SemaphoreType.DMA,
        ],
    )(_sc_cnt_body)
    sc_edge = functools.partial(
        pl.kernel,
        out_type=jax.ShapeDtypeStruct((NC, N_PAD, D), jnp.float32),
        mesh=mesh,
        scratch_types=[
            pltpu.VMEM((EPW,), jnp.int32),
            pltpu.VMEM((1, CH_E), jnp.int32),
            pltpu.VMEM((1, CH_E), jnp.int32),
            pltpu.VMEM((1, CH_E), jnp.int32),
            pltpu.VMEM((1, CH_E), jnp.int32),
            pltpu.VMEM((CH_E, D), jnp.float32),
            pltpu.VMEM((CH_E, D), jnp.float32),
            pltpu.VMEM_SHARED((N_PAD, D), jnp.float32),
            pltpu.SemaphoreType.DMA,
            pltpu.SemaphoreType.DMA,
        ],
    )(_sc_edge_body)
    return sc_cnt, sc_edge


BLK = 1024
GRID = N_PAD // BLK


def _leaky(x):
    return jnp.where(x >= 0, x, RRELU_SLOPE * x)


def _tc_pre_body(x_ref, wn_ref, ws_ref, ht_ref, hw_ref, lp_ref):
    x = x_ref[...]
    n = jnp.sqrt(jnp.sum(x * x, axis=-1, keepdims=True))
    n = jnp.maximum(n, 1e-10)
    y = jnp.clip(SQRT_C * n, -1.0 + 1e-7, 1.0 - 1e-7)
    at = 0.5 * jnp.log((1.0 + y) / (1.0 - y))
    ht = at * x / (SQRT_C * n)
    ht_ref[...] = ht
    hw_ref[...] = jnp.dot(ht, wn_ref[...], preferred_element_type=jnp.float32)
    lp_ref[...] = jnp.dot(ht, ws_ref[...], preferred_element_type=jnp.float32)


def _tc_combine_body(p0_ref, p1_ref, cnt_ref, rel_ref, wn_ref, lp_ref,
                     wn2_ref, ws2_ref, hw2_ref, lp2_ref):
    relw = jnp.dot(rel_ref[...], wn_ref[...], preferred_element_type=jnp.float32)
    cnt = cnt_ref[...]
    aggrel = jnp.dot(cnt, relw, preferred_element_type=jnp.float32)
    deg = jnp.maximum(jnp.sum(cnt, axis=-1, keepdims=True), 1.0)
    pre = (p0_ref[...] + p1_ref[...] + aggrel) / deg + lp_ref[...]
    ht = _leaky(pre)
    hw2_ref[...] = jnp.dot(ht, wn2_ref[...], preferred_element_type=jnp.float32)
    lp2_ref[...] = jnp.dot(ht, ws2_ref[...], preferred_element_type=jnp.float32)


def _tc_final_body(p0_ref, p1_ref, cnt_ref, rel_ref, wn_ref, lp_ref, out_ref):
    relw = jnp.dot(rel_ref[...], wn_ref[...], preferred_element_type=jnp.float32)
    cnt = cnt_ref[...]
    aggrel = jnp.dot(cnt, relw, preferred_element_type=jnp.float32)
    deg = jnp.maximum(jnp.sum(cnt, axis=-1, keepdims=True), 1.0)
    pre = (p0_ref[...] + p1_ref[...] + aggrel) / deg + lp_ref[...]
    ht = _leaky(pre)
    n = jnp.sqrt(jnp.sum(ht * ht, axis=-1, keepdims=True))
    n = jnp.maximum(n, 1e-10)
    out_ref[...] = jnp.tanh(SQRT_C * n) * ht / (SQRT_C * n)


_row_spec = pl.BlockSpec((BLK, D), lambda i: (i, 0))
_cnt_spec = pl.BlockSpec((BLK, R_PAD), lambda i: (i, 0))
_full_spec = pl.BlockSpec((D, D), lambda i: (0, 0))
_rel_spec = pl.BlockSpec((R_PAD, D), lambda i: (0, 0))
_rowD = jax.ShapeDtypeStruct((N_PAD, D), jnp.float32)


def _tc_pre(x, wn, ws):
    return pl.pallas_call(
        _tc_pre_body,
        grid=(GRID,),
        in_specs=[_row_spec, _full_spec, _full_spec],
        out_specs=[_row_spec, _row_spec, _row_spec],
        out_shape=[_rowD, _rowD, _rowD],
    )(x, wn, ws)


def _tc_combine(p0, p1, cnt, rel, wn, lp, wn2, ws2):
    return pl.pallas_call(
        _tc_combine_body,
        grid=(GRID,),
        in_specs=[_row_spec, _row_spec, _cnt_spec, _rel_spec, _full_spec,
                  _row_spec, _full_spec, _full_spec],
        out_specs=[_row_spec, _row_spec],
        out_shape=[_rowD, _rowD],
    )(p0, p1, cnt, rel, wn, lp, wn2, ws2)


def _tc_final(p0, p1, cnt, rel, wn, lp):
    return pl.pallas_call(
        _tc_final_body,
        grid=(GRID,),
        in_specs=[_row_spec, _row_spec, _cnt_spec, _rel_spec, _full_spec,
                  _row_spec],
        out_specs=_row_spec,
        out_shape=_rowD,
    )(p0, p1, cnt, rel, wn, lp)


def kernel(init_ent_emb, node_id, edge_index, edge_type, init_rel_emb,
           W_neigh, W_self):
    h = jnp.take(init_ent_emb, node_id, axis=0)
    h_pad = jnp.pad(h, ((0, N_PAD - N), (0, 0)))
    rel_pad = jnp.pad(init_rel_emb, ((0, R_PAD - R), (0, 0)))

    src = jnp.pad(edge_index[0], (0, E_PAD - E))
    dst = jnp.pad(edge_index[1], (0, E_PAD - E), constant_values=N)
    et = jnp.pad(edge_type, (0, E_PAD - E))

    dst2 = dst.reshape(NS, EPT)
    et2 = et.reshape(NS, EPT)
    pack2 = (src | (dst << 14)).reshape(NW, EPW)

    zeros_flat = jnp.zeros((CNT_W,), jnp.float32)
    zeros_rows = jnp.zeros((N_PAD, D), jnp.float32)
    _sc_cnt, _sc_edge = _sc_kernels()
    cnt4 = _sc_cnt(dst2, et2, zeros_flat)
    cnt = jnp.concatenate(
        [cnt4[0, 0].reshape(N_PAD, R_SC), cnt4[0, 1].reshape(N_PAD, R_SC),
         cnt4[1, 0].reshape(N_PAD, R_SC), cnt4[1, 1].reshape(N_PAD, R_SC)],
        axis=1)
    cnt = jnp.pad(cnt, ((0, 0), (0, R_PAD - 4 * R_SC)))

    h_tan, hw0, lp0 = _tc_pre(h_pad, W_neigh[0], W_self[0])
    part0 = _sc_edge(hw0, pack2, zeros_rows)
    hw1, lp1 = _tc_combine(part0[0], part0[1], cnt, rel_pad, W_neigh[0], lp0,
                           W_neigh[1], W_self[1])
    part1 = _sc_edge(hw1, pack2, zeros_rows)
    out = _tc_final(part1[0], part1[1], cnt, rel_pad, W_neigh[1], lp1)
    return out[:N]
